# baseline (speedup 1.0000x reference)
import functools

import jax
import jax.numpy as jnp
from jax import lax
from jax.experimental import pallas as pl
from jax.experimental.pallas import tpu as pltpu
from jax.experimental.pallas import tpu_sc as plsc

N = 10000
E = 320000
D = 128

NC = 2
NS = 16
CHUNK = 80
EPT = E // (NC * NS)
ITERS = EPT // CHUNK
NBLK = N // CHUNK
BLK_ROUNDS = -(-NBLK // NS)


def _init_acc(sh_ref, zbuf, s):
    for k in range(BLK_ROUNDS):
        b = k * NS + s

        @pl.when(b < NBLK)
        def _():
            pltpu.sync_copy(zbuf, sh_ref.at[pl.ds(b * CHUNK, CHUNK)])


def _drain_acc(sh_ref, bbuf, out_hbm, c, s):
    for k in range(BLK_ROUNDS):
        b = k * NS + s

        @pl.when(b < NBLK)
        def _():
            pltpu.sync_copy(sh_ref.at[pl.ds(b * CHUNK, CHUNK)], bbuf)
            pltpu.sync_copy(bbuf, out_hbm.at[pl.ds(c * N + b * CHUNK, CHUNK)])


def _sc_body(x_hbm, ea_hbm, ei4_hbm, zeros_hbm, ones_hbm,
             outa_hbm, outd_hbm,
             a_sh, idx_v, rows_v,
             sem_g0, sem_g1, sem_g2, sem_s0, sem_s1, sem_s2):
    c = lax.axis_index("c")
    s = lax.axis_index("s")
    tid = c * NS + s
    base = tid * EPT

    slots = [rows_v.at[pl.ds(k * CHUNK, CHUNK)] for k in range(3)]
    ea_s = slots[0]

    pltpu.sync_copy(zeros_hbm, slots[0])
    _init_acc(a_sh, slots[0], s)
    pltpu.sync_copy(ei4_hbm.at[1, tid], idx_v.at[pl.ds(0, ITERS)])
    plsc.subcore_barrier()

    def fire_src(j, slot, sem):
        pltpu.async_copy(ei4_hbm.at[0, tid, pl.ds(j, 1)],
                         idx_v.at[pl.ds(slot, 1)], sem)

    def fire_gather(k, sg):
        pltpu.async_copy(x_hbm.at[idx_v.at[ITERS + k]], slots[k], sg)

    def wait_ld(buf, sem):
        pltpu.make_async_copy(ea_hbm.at[pl.ds(0, CHUNK)], buf, sem).wait()

    def wait_src(k, sem):
        pltpu.make_async_copy(ei4_hbm.at[0, 0, pl.ds(0, 1)],
                              idx_v.at[pl.ds(ITERS + k, 1)], sem).wait()

    def scatter(j, k):
        pltpu.sync_copy(slots[k], a_sh.at[idx_v.at[j]], add=True)
        pltpu.sync_copy(ea_hbm.at[pl.ds(base + j * CHUNK, CHUNK)], slots[k])
        pltpu.sync_copy(slots[k], a_sh.at[idx_v.at[j]], add=True)

    sem_g = [sem_g0, sem_g1, sem_g2]
    sem_s = [sem_s0, sem_s1, sem_s2]

    fire_src(0, ITERS + 0, sem_s[0])
    fire_src(1, ITERS + 1, sem_s[1])
    fire_src(2, ITERS + 2, sem_s[2])
    wait_src(0, sem_s[0])
    fire_gather(0, sem_g[0])
    wait_src(1, sem_s[1])
    fire_gather(1, sem_g[1])

    def sub(j, q):
        p = (q + 2) % 3
        wait_ld(slots[q], sem_g[q])

        @pl.when(j + 3 < ITERS)
        def _():
            fire_src(j + 3, ITERS + q, sem_s[q])

        @pl.when(j + 2 < ITERS)
        def _():
            wait_src(p, sem_s[p])
            fire_gather(p, sem_g[p])

        scatter(j, q)

    def triple(t, carry):
        j0 = 3 * t
        sub(j0, 0)
        sub(j0 + 1, 1)
        sub(j0 + 2, 2)
        return carry

    lax.fori_loop(0, ITERS // 3, triple, 0)
    sub(ITERS - 2, (ITERS - 2) % 3)
    sub(ITERS - 1, (ITERS - 1) % 3)
    plsc.subcore_barrier()
    _drain_acc(a_sh, slots[0], outa_hbm, c, s)

    pltpu.sync_copy(zeros_hbm, slots[1])
    _init_acc(a_sh, slots[1], s)
    pltpu.sync_copy(ones_hbm, ea_s)
    plsc.subcore_barrier()

    def dstep(j, carry):
        pltpu.sync_copy(ea_s, a_sh.at[idx_v.at[j]], add=True)
        return carry

    lax.fori_loop(0, ITERS, dstep, 0)
    plsc.subcore_barrier()
    _drain_acc(a_sh, slots[1], outd_hbm, c, s)


@jax.jit
def _sc_aggregate(x, edge_attr, ei4):
    zeros = jnp.zeros((CHUNK, D), jnp.float32)
    ones = jnp.ones((CHUNK, D), jnp.float32)
    mesh = plsc.VectorSubcoreMesh(core_axis_name="c", subcore_axis_name="s")
    fn = functools.partial(
        pl.kernel,
        _sc_body,
        out_type=[
            jax.ShapeDtypeStruct((NC * N, D), jnp.float32),
            jax.ShapeDtypeStruct((NC * N, D), jnp.float32),
        ],
        mesh=mesh,
        scratch_types=[
            pltpu.VMEM_SHARED((N, D), jnp.float32),
            pltpu.VMEM((ITERS + 3, CHUNK), jnp.int32),
            pltpu.VMEM((3 * CHUNK, D), jnp.float32),
            pltpu.SemaphoreType.DMA,
            pltpu.SemaphoreType.DMA,
            pltpu.SemaphoreType.DMA,
            pltpu.SemaphoreType.DMA,
            pltpu.SemaphoreType.DMA,
            pltpu.SemaphoreType.DMA,
        ],
    )()
    a_flat, d_flat = fn(x, edge_attr, ei4, zeros, ones)
    return a_flat, d_flat


def _tc_body(a0_ref, a1_ref, d0_ref, d1_ref, w_ref, b_ref, out_ref):
    a = a0_ref[...] + a1_ref[...]
    deg = d0_ref[:, :1] + d1_ref[:, :1]
    norm = jnp.where(deg > 0.0, 1.0 / jnp.sqrt(deg), 0.0)
    out_ref[...] = (
        jnp.dot(a * norm, w_ref[...], preferred_element_type=jnp.float32)
        + b_ref[...]
    )


@jax.jit
def _tc_finish(a_flat, d_flat, weights, h_bias):
    bn = 400
    off1 = N // bn
    return pl.pallas_call(
        _tc_body,
        grid=(N // bn,),
        in_specs=[
            pl.BlockSpec((bn, D), lambda i: (i, 0)),
            pl.BlockSpec((bn, D), lambda i: (i + off1, 0)),
            pl.BlockSpec((bn, D), lambda i: (i, 0)),
            pl.BlockSpec((bn, D), lambda i: (i + off1, 0)),
            pl.BlockSpec((D, D), lambda i: (0, 0)),
            pl.BlockSpec((1, D), lambda i: (0, 0)),
        ],
        out_specs=pl.BlockSpec((bn, D), lambda i: (i, 0)),
        out_shape=jax.ShapeDtypeStruct((N, D), jnp.float32),
    )(a_flat, a_flat, d_flat, d_flat, weights, h_bias.reshape(1, D))


def kernel(x, edge_attr, weights, h_bias, edge_index):
    ei4 = edge_index.reshape(2, NC * NS, ITERS, CHUNK)
    a_flat, d_flat = _sc_aggregate(x, edge_attr, ei4)
    return _tc_finish(a_flat, d_flat, weights, h_bias)
